# baseline (speedup 1.0000x reference)
import functools

import jax
import jax.numpy as jnp
from jax import lax
from jax.scipy.special import gammaln, digamma
from jax.experimental import pallas as pl
from jax.experimental.pallas import tpu as pltpu
from jax.experimental.pallas import tpu_sc as plsc

NV = 4096
NH = 65536
ND = 102400
NN = NV * NV

NC, NS = 2, 16
NW = NC * NS
HEPW = NH // NW
DEPW = ND // NW
NSTR = 4
HH = HEPW // NSTR
DH = DEPW // NSTR


def _softplus(x):
    return jnp.maximum(x, 0.0) + jnp.log1p(jnp.exp(-jnp.abs(x)))


def _prep_body(th, ar, br, wd, kt, hr, hc, dr, dc,
               hw_o, pk_o, hf_o, df_o, kl_o):
    u32 = jnp.uint32
    hw_o[...] = _softplus(th[...])
    a = _softplus(ar[...]) + 0.001
    b = _softplus(br[...]) + 0.001
    piv = a / (a + b)
    hf_o[...] = hr[...] * NV + hc[...]
    df_o[...] = dr[...] * NV + dc[...]
    wb = lax.bitcast_convert_type(wd[...], u32) + u32(0x8000)
    pb = lax.bitcast_convert_type(piv, u32) + u32(0x8000)
    packed = (pb & u32(0xFFFF0000)) | (wb >> u32(16))
    pk_o[...] = lax.bitcast_convert_type(packed, jnp.int32)
    kl_o[0] = jnp.sum(kt[...])


def _prep(theta, a_raw, b_raw, wd, kl_terms, hr, hc, dr, dc):
    f32 = jnp.float32
    i32 = jnp.int32
    return pl.pallas_call(
        _prep_body,
        out_shape=(
            jax.ShapeDtypeStruct((NH // 128, 128), f32),
            jax.ShapeDtypeStruct((ND // 128, 128), i32),
            jax.ShapeDtypeStruct((NH // 128, 128), i32),
            jax.ShapeDtypeStruct((ND // 128, 128), i32),
            jax.ShapeDtypeStruct((1,), f32),
        ),
        out_specs=(
            pl.BlockSpec(),
            pl.BlockSpec(),
            pl.BlockSpec(),
            pl.BlockSpec(),
            pl.BlockSpec(memory_space=pltpu.MemorySpace.SMEM),
        ),
    )(theta, a_raw, b_raw, wd, kl_terms, hr, hc, dr, dc)


def _sc_scatter_body(hf, hw, df, pk, m1, m2, *scr):
    hi = scr[0:NSTR]
    hv = scr[NSTR:2 * NSTR]
    di = scr[2 * NSTR:3 * NSTR]
    dv = scr[3 * NSTR:4 * NSTR]
    sem = scr[4 * NSTR]
    c = lax.axis_index("c")
    s = lax.axis_index("s")
    wid = s * NC + c
    hb = pl.multiple_of(wid * HEPW, HEPW)
    db = pl.multiple_of(wid * DEPW, DEPW)
    for j in range(NSTR):
        pltpu.sync_copy(hf.at[pl.ds(hb + j * HH, HH)], hi[j])
        pltpu.sync_copy(hw.at[pl.ds(hb + j * HH, HH)], hv[j])
        pltpu.sync_copy(df.at[pl.ds(db + j * DH, DH)], di[j])
        pltpu.sync_copy(pk.at[pl.ds(db + j * DH, DH)], dv[j])
    cps = []
    for j in range(NSTR):
        cps.append(pltpu.async_copy(hv[j], m1.at[hi[j]], sem))
        cps.append(pltpu.async_copy(dv[j], m2.at[di[j]], sem))
    for cp in cps:
        cp.wait()


@functools.cache
def _sc_scatter_kernel():
    return pl.kernel(
        _sc_scatter_body,
        out_type=(),
        mesh=plsc.VectorSubcoreMesh(core_axis_name="c", subcore_axis_name="s",
                                    num_cores=NC, num_subcores=NS),
        scratch_types=(
            [pltpu.VMEM((HH,), jnp.int32)] * NSTR
            + [pltpu.VMEM((HH,), jnp.float32)] * NSTR
            + [pltpu.VMEM((DH,), jnp.int32)] * NSTR
            + [pltpu.VMEM((DH,), jnp.int32)] * NSTR
            + [pltpu.SemaphoreType.DMA]
        ),
    )


_MUL_ROWS = 256


def _mul_body(m1, m2, out):
    i32 = jnp.int32
    w1 = m1[...]
    v = m2[...]
    w_f = lax.bitcast_convert_type(v << i32(16), jnp.float32)
    pi_f = lax.bitcast_convert_type(v & i32(-65536), jnp.float32)
    out[...] = (jnp.where(w1 != 0.0, 1.0, 0.0) + pi_f) * (w1 + w_f)


def _mul(m1, m2):
    grid = NV // _MUL_ROWS
    spec = pl.BlockSpec((_MUL_ROWS, NV), lambda i: (i, 0))
    return pl.pallas_call(
        _mul_body,
        grid=(grid,),
        in_specs=[spec, spec],
        out_specs=spec,
        out_shape=jax.ShapeDtypeStruct((NV, NV), jnp.float32),
    )(m1, m2)


def kernel(theta_hard, w_disc, a_raw, b_raw, prior_a, prior_b,
           hard_idx, disc_idx):
    i32 = jnp.int32
    hr = hard_idx[:, 0].astype(i32).reshape(NH // 128, 128)
    hc = hard_idx[:, 1].astype(i32).reshape(NH // 128, 128)
    dr = disc_idx[:, 0].astype(i32).reshape(ND // 128, 128)
    dc = disc_idx[:, 1].astype(i32).reshape(ND // 128, 128)
    theta = theta_hard.reshape(NH // 128, 128)
    ar = a_raw.reshape(ND // 128, 128)
    br = b_raw.reshape(ND // 128, 128)

    a = jax.nn.softplus(a_raw) + 0.001
    b = jax.nn.softplus(b_raw) + 0.001
    kt = (gammaln(prior_a) + gammaln(prior_b) - gammaln(prior_a + prior_b)
          - gammaln(a) - gammaln(b) + gammaln(a + b)
          + (a - prior_a) * digamma(a)
          + (b - prior_b) * digamma(b)
          + (prior_a + prior_b - a - b) * digamma(a + b))

    hw, pk, hf, df, kl = _prep(theta, ar, br, w_disc.reshape(ND // 128, 128),
                               kt.reshape(ND // 128, 128), hr, hc, dr, dc)

    m1 = jax.new_ref(jnp.zeros((NN,), jnp.float32))
    m2 = jax.new_ref(jnp.zeros((NN,), jnp.int32))
    _sc_scatter_kernel()(hf.reshape(NH), hw.reshape(NH), df.reshape(ND),
                         pk.reshape(ND), m1, m2)

    eff = _mul(m1[...].reshape(NV, NV), m2[...].reshape(NV, NV))
    return eff, kl[0]


# speedup vs baseline: 1.0047x; 1.0047x over previous
import functools

import jax
import jax.numpy as jnp
from jax import lax
from jax.scipy.special import gammaln, digamma
from jax.experimental import pallas as pl
from jax.experimental.pallas import tpu as pltpu
from jax.experimental.pallas import tpu_sc as plsc

NV = 4096
NH = 65536
ND = 102400
NN = NV * NV

NC, NS = 2, 16
NW = NC * NS
HEPW = NH // NW
DEPW = ND // NW
NSTR = 2
HH = HEPW // NSTR
DH = DEPW // NSTR


def _softplus(x):
    return jnp.maximum(x, 0.0) + jnp.log1p(jnp.exp(-jnp.abs(x)))


def _prep_body(th, ar, br, wd, kt, hr, hc, dr, dc,
               hw_o, pk_o, hf_o, df_o, kl_o):
    u32 = jnp.uint32
    hw_o[...] = _softplus(th[...])
    a = _softplus(ar[...]) + 0.001
    b = _softplus(br[...]) + 0.001
    piv = a / (a + b)
    hf_o[...] = hr[...] * NV + hc[...]
    df_o[...] = dr[...] * NV + dc[...]
    wb = lax.bitcast_convert_type(wd[...], u32) + u32(0x8000)
    pb = lax.bitcast_convert_type(piv, u32) + u32(0x8000)
    packed = (pb & u32(0xFFFF0000)) | (wb >> u32(16))
    pk_o[...] = lax.bitcast_convert_type(packed, jnp.int32)
    kl_o[0] = jnp.sum(kt[...])


def _prep(theta, a_raw, b_raw, wd, kl_terms, hr, hc, dr, dc):
    f32 = jnp.float32
    i32 = jnp.int32
    return pl.pallas_call(
        _prep_body,
        out_shape=(
            jax.ShapeDtypeStruct((NH // 128, 128), f32),
            jax.ShapeDtypeStruct((ND // 128, 128), i32),
            jax.ShapeDtypeStruct((NH // 128, 128), i32),
            jax.ShapeDtypeStruct((ND // 128, 128), i32),
            jax.ShapeDtypeStruct((1,), f32),
        ),
        out_specs=(
            pl.BlockSpec(),
            pl.BlockSpec(),
            pl.BlockSpec(),
            pl.BlockSpec(),
            pl.BlockSpec(memory_space=pltpu.MemorySpace.SMEM),
        ),
    )(theta, a_raw, b_raw, wd, kl_terms, hr, hc, dr, dc)


def _sc_scatter_body(hf, hw, df, pk, m1, m2, *scr):
    hi = scr[0:NSTR]
    hv = scr[NSTR:2 * NSTR]
    di = scr[2 * NSTR:3 * NSTR]
    dv = scr[3 * NSTR:4 * NSTR]
    sem = scr[4 * NSTR]
    c = lax.axis_index("c")
    s = lax.axis_index("s")
    wid = s * NC + c
    hb = pl.multiple_of(wid * HEPW, HEPW)
    db = pl.multiple_of(wid * DEPW, DEPW)
    for j in range(NSTR):
        pltpu.sync_copy(hf.at[pl.ds(hb + j * HH, HH)], hi[j])
        pltpu.sync_copy(hw.at[pl.ds(hb + j * HH, HH)], hv[j])
        pltpu.sync_copy(df.at[pl.ds(db + j * DH, DH)], di[j])
        pltpu.sync_copy(pk.at[pl.ds(db + j * DH, DH)], dv[j])
    cps = []
    for j in range(NSTR):
        cps.append(pltpu.async_copy(hv[j], m1.at[hi[j]], sem))
        cps.append(pltpu.async_copy(dv[j], m2.at[di[j]], sem))
    for cp in cps:
        cp.wait()


@functools.cache
def _sc_scatter_kernel():
    return pl.kernel(
        _sc_scatter_body,
        out_type=(),
        mesh=plsc.VectorSubcoreMesh(core_axis_name="c", subcore_axis_name="s",
                                    num_cores=NC, num_subcores=NS),
        scratch_types=(
            [pltpu.VMEM((HH,), jnp.int32)] * NSTR
            + [pltpu.VMEM((HH,), jnp.float32)] * NSTR
            + [pltpu.VMEM((DH,), jnp.int32)] * NSTR
            + [pltpu.VMEM((DH,), jnp.int32)] * NSTR
            + [pltpu.SemaphoreType.DMA]
        ),
    )


_MUL_ROWS = 512


def _mul_body(m1, m2, out):
    i32 = jnp.int32
    w1 = m1[...]
    v = m2[...]
    w_f = lax.bitcast_convert_type(v << i32(16), jnp.float32)
    pi_f = lax.bitcast_convert_type(v & i32(-65536), jnp.float32)
    out[...] = (jnp.where(w1 != 0.0, 1.0, 0.0) + pi_f) * (w1 + w_f)


def _mul(m1, m2):
    grid = NV // _MUL_ROWS
    spec = pl.BlockSpec((_MUL_ROWS, NV), lambda i: (i, 0))
    return pl.pallas_call(
        _mul_body,
        grid=(grid,),
        in_specs=[spec, spec],
        out_specs=spec,
        out_shape=jax.ShapeDtypeStruct((NV, NV), jnp.float32),
    )(m1, m2)


def kernel(theta_hard, w_disc, a_raw, b_raw, prior_a, prior_b,
           hard_idx, disc_idx):
    i32 = jnp.int32
    hr = hard_idx[:, 0].astype(i32).reshape(NH // 128, 128)
    hc = hard_idx[:, 1].astype(i32).reshape(NH // 128, 128)
    dr = disc_idx[:, 0].astype(i32).reshape(ND // 128, 128)
    dc = disc_idx[:, 1].astype(i32).reshape(ND // 128, 128)
    theta = theta_hard.reshape(NH // 128, 128)
    ar = a_raw.reshape(ND // 128, 128)
    br = b_raw.reshape(ND // 128, 128)

    a = jax.nn.softplus(a_raw) + 0.001
    b = jax.nn.softplus(b_raw) + 0.001
    kt = (gammaln(prior_a) + gammaln(prior_b) - gammaln(prior_a + prior_b)
          - gammaln(a) - gammaln(b) + gammaln(a + b)
          + (a - prior_a) * digamma(a)
          + (b - prior_b) * digamma(b)
          + (prior_a + prior_b - a - b) * digamma(a + b))

    hw, pk, hf, df, kl = _prep(theta, ar, br, w_disc.reshape(ND // 128, 128),
                               kt.reshape(ND // 128, 128), hr, hc, dr, dc)

    m1 = jax.new_ref(jnp.zeros((NN,), jnp.float32))
    m2 = jax.new_ref(jnp.zeros((NN,), jnp.int32))
    _sc_scatter_kernel()(hf.reshape(NH), hw.reshape(NH), df.reshape(ND),
                         pk.reshape(ND), m1, m2)

    eff = _mul(m1[...].reshape(NV, NV), m2[...].reshape(NV, NV))
    return eff, kl[0]


# speedup vs baseline: 1.2359x; 1.2301x over previous
import functools

import jax
import jax.numpy as jnp
from jax import lax
from jax.scipy.special import gammaln, digamma
from jax.experimental import pallas as pl
from jax.experimental.pallas import tpu as pltpu
from jax.experimental.pallas import tpu_sc as plsc

NV = 4096
NH = 65536
ND = 102400
NN = NV * NV

NC, NS = 2, 16
NW = NC * NS
HEPW = NH // NW
DEPW = ND // NW
NSTR = 2
HH = HEPW // NSTR
DH = DEPW // NSTR


def _softplus(x):
    return jnp.maximum(x, 0.0) + jnp.log1p(jnp.exp(-jnp.abs(x)))


def _tiled_flat(r, c):
    return (((r >> 3) * (NV // 128) + (c >> 7)) * 1024
            + (r & 7) * 128 + (c & 127))


def _prep_body(th, ar, br, wd, hr, hc, dr, dc,
               hw_o, pk_o, hf_o, df_o):
    u32 = jnp.uint32
    hw_o[...] = _softplus(th[...])
    a = _softplus(ar[...]) + 0.001
    b = _softplus(br[...]) + 0.001
    piv = a / (a + b)
    hf_o[...] = _tiled_flat(hr[...], hc[...])
    df_o[...] = _tiled_flat(dr[...], dc[...])
    wb = lax.bitcast_convert_type(wd[...], u32) + u32(0x8000)
    pb = lax.bitcast_convert_type(piv, u32) + u32(0x8000)
    packed = (pb & u32(0xFFFF0000)) | (wb >> u32(16))
    pk_o[...] = lax.bitcast_convert_type(packed, jnp.int32)


def _prep(theta, a_raw, b_raw, wd, hr, hc, dr, dc):
    f32 = jnp.float32
    i32 = jnp.int32
    return pl.pallas_call(
        _prep_body,
        out_shape=(
            jax.ShapeDtypeStruct((NH // 128, 128), f32),
            jax.ShapeDtypeStruct((ND // 128, 128), i32),
            jax.ShapeDtypeStruct((NH // 128, 128), i32),
            jax.ShapeDtypeStruct((ND // 128, 128), i32),
        ),
    )(theta, a_raw, b_raw, wd, hr, hc, dr, dc)


def _sc_scatter_body(hf, hw, df, pk, m1, m2, *scr):
    hi = scr[0:NSTR]
    hv = scr[NSTR:2 * NSTR]
    di = scr[2 * NSTR:3 * NSTR]
    dv = scr[3 * NSTR:4 * NSTR]
    sem = scr[4 * NSTR]
    c = lax.axis_index("c")
    s = lax.axis_index("s")
    wid = s * NC + c
    hb = pl.multiple_of(wid * HEPW, HEPW)
    db = pl.multiple_of(wid * DEPW, DEPW)
    for j in range(NSTR):
        pltpu.sync_copy(hf.at[pl.ds(hb + j * HH, HH)], hi[j])
        pltpu.sync_copy(hw.at[pl.ds(hb + j * HH, HH)], hv[j])
        pltpu.sync_copy(df.at[pl.ds(db + j * DH, DH)], di[j])
        pltpu.sync_copy(pk.at[pl.ds(db + j * DH, DH)], dv[j])
    cps = []
    for j in range(NSTR):
        cps.append(pltpu.async_copy(hv[j], m1.at[hi[j]], sem))
        cps.append(pltpu.async_copy(dv[j], m2.at[di[j]], sem))
    for cp in cps:
        cp.wait()


@functools.cache
def _sc_scatter_kernel():
    return pl.kernel(
        _sc_scatter_body,
        out_type=(),
        mesh=plsc.VectorSubcoreMesh(core_axis_name="c", subcore_axis_name="s",
                                    num_cores=NC, num_subcores=NS),
        scratch_types=(
            [pltpu.VMEM((HH,), jnp.int32)] * NSTR
            + [pltpu.VMEM((HH,), jnp.float32)] * NSTR
            + [pltpu.VMEM((DH,), jnp.int32)] * NSTR
            + [pltpu.VMEM((DH,), jnp.int32)] * NSTR
            + [pltpu.SemaphoreType.DMA]
        ),
    )


_MUL_ROWS = 256
_TR = _MUL_ROWS // 8
_TC = NV // 128
_BLK = _MUL_ROWS * NV


def _mul_body(m1, m2, kt, out, kl_o):
    i32 = jnp.int32
    w1 = m1[...]
    v = m2[...]
    w_f = lax.bitcast_convert_type(v << i32(16), jnp.float32)
    pi_f = lax.bitcast_convert_type(v & i32(-65536), jnp.float32)
    eff = (jnp.where(w1 != 0.0, 1.0, 0.0) + pi_f) * (w1 + w_f)
    out[...] = (eff.reshape(_TR, _TC, 8, 128)
                .transpose(0, 2, 1, 3)
                .reshape(_MUL_ROWS, NV))

    @pl.when(pl.program_id(0) == 0)
    def _():
        kl_o[0] = jnp.sum(kt[...])


def _mul(m1, m2, kt):
    grid = NN // _BLK
    flat_spec = pl.BlockSpec((_BLK,), lambda i: (i,))
    return pl.pallas_call(
        _mul_body,
        grid=(grid,),
        in_specs=[flat_spec, flat_spec,
                  pl.BlockSpec((ND // 128, 128), lambda i: (0, 0))],
        out_specs=(
            pl.BlockSpec((_MUL_ROWS, NV), lambda i: (i, 0)),
            pl.BlockSpec(memory_space=pltpu.MemorySpace.SMEM),
        ),
        out_shape=(
            jax.ShapeDtypeStruct((NV, NV), jnp.float32),
            jax.ShapeDtypeStruct((1,), jnp.float32),
        ),
    )(m1, m2, kt)


def kernel(theta_hard, w_disc, a_raw, b_raw, prior_a, prior_b,
           hard_idx, disc_idx):
    i32 = jnp.int32
    hr = hard_idx[:, 0].astype(i32).reshape(NH // 128, 128)
    hc = hard_idx[:, 1].astype(i32).reshape(NH // 128, 128)
    dr = disc_idx[:, 0].astype(i32).reshape(ND // 128, 128)
    dc = disc_idx[:, 1].astype(i32).reshape(ND // 128, 128)
    theta = theta_hard.reshape(NH // 128, 128)
    ar = a_raw.reshape(ND // 128, 128)
    br = b_raw.reshape(ND // 128, 128)

    a = jax.nn.softplus(a_raw) + 0.001
    b = jax.nn.softplus(b_raw) + 0.001
    kt = (gammaln(prior_a) + gammaln(prior_b) - gammaln(prior_a + prior_b)
          - gammaln(a) - gammaln(b) + gammaln(a + b)
          + (a - prior_a) * digamma(a)
          + (b - prior_b) * digamma(b)
          + (prior_a + prior_b - a - b) * digamma(a + b))

    hw, pk, hf, df = _prep(theta, ar, br, w_disc.reshape(ND // 128, 128),
                           hr, hc, dr, dc)

    m1 = jax.new_ref(jnp.zeros((NN,), jnp.float32))
    m2 = jax.new_ref(jnp.zeros((NN,), jnp.int32))
    _sc_scatter_kernel()(hf.reshape(NH), hw.reshape(NH), df.reshape(ND),
                         pk.reshape(ND), m1, m2)

    eff, kl = _mul(m1[...], m2[...], kt.reshape(ND // 128, 128))
    return eff, kl[0]


# speedup vs baseline: 1.2432x; 1.0058x over previous
import functools

import jax
import jax.numpy as jnp
from jax import lax
from jax.scipy.special import gammaln, digamma
from jax.experimental import pallas as pl
from jax.experimental.pallas import tpu as pltpu
from jax.experimental.pallas import tpu_sc as plsc

NV = 4096
NH = 65536
ND = 102400
NN = NV * NV

NC, NS = 2, 16
NW = NC * NS
HEPW = NH // NW
DEPW = ND // NW
NSTR = 2
HH = HEPW // NSTR
DH = DEPW // NSTR


def _softplus(x):
    return jnp.maximum(x, 0.0) + jnp.log1p(jnp.exp(-jnp.abs(x)))


def _tiled_flat(r, c):
    return (((r >> 3) * (NV // 128) + (c >> 7)) * 1024
            + (r & 7) * 128 + (c & 127))


def _prep_body(th, ar, br, wd, hr, hc, dr, dc,
               hw_o, pk_o, hf_o, df_o):
    u32 = jnp.uint32
    hw_o[...] = _softplus(th[...])
    a = _softplus(ar[...]) + 0.001
    b = _softplus(br[...]) + 0.001
    piv = a / (a + b)
    hf_o[...] = _tiled_flat(hr[...], hc[...])
    df_o[...] = _tiled_flat(dr[...], dc[...])
    wb = lax.bitcast_convert_type(wd[...], u32) + u32(0x8000)
    pb = lax.bitcast_convert_type(piv, u32) + u32(0x8000)
    packed = (pb & u32(0xFFFF0000)) | (wb >> u32(16))
    pk_o[...] = lax.bitcast_convert_type(packed, jnp.int32)


def _prep(theta, a_raw, b_raw, wd, hr, hc, dr, dc):
    f32 = jnp.float32
    i32 = jnp.int32
    return pl.pallas_call(
        _prep_body,
        out_shape=(
            jax.ShapeDtypeStruct((NH // 128, 128), f32),
            jax.ShapeDtypeStruct((ND // 128, 128), i32),
            jax.ShapeDtypeStruct((NH // 128, 128), i32),
            jax.ShapeDtypeStruct((ND // 128, 128), i32),
        ),
    )(theta, a_raw, b_raw, wd, hr, hc, dr, dc)


def _sc_scatter1_body(idx_hbm, val_hbm, m, *scr):
    n = (len(scr) - 1) // 2
    ivs = scr[0:n]
    vvs = scr[n:2 * n]
    sem = scr[2 * n]
    c = lax.axis_index("c")
    s = lax.axis_index("s")
    wid = s * NC + c
    epw = idx_hbm.shape[0] // NW
    ch = epw // n
    base = pl.multiple_of(wid * epw, epw)
    for j in range(n):
        pltpu.sync_copy(idx_hbm.at[pl.ds(base + j * ch, ch)], ivs[j])
        pltpu.sync_copy(val_hbm.at[pl.ds(base + j * ch, ch)], vvs[j])
    cps = [pltpu.async_copy(vvs[j], m.at[ivs[j]], sem) for j in range(n)]
    for cp in cps:
        cp.wait()


@functools.cache
def _sc_scatter1(n_edges, val_dtype):
    ch = n_edges // NW // NSTR
    return pl.kernel(
        _sc_scatter1_body,
        out_type=(),
        mesh=plsc.VectorSubcoreMesh(core_axis_name="c", subcore_axis_name="s",
                                    num_cores=NC, num_subcores=NS),
        scratch_types=(
            [pltpu.VMEM((ch,), jnp.int32)] * NSTR
            + [pltpu.VMEM((ch,), val_dtype)] * NSTR
            + [pltpu.SemaphoreType.DMA]
        ),
    )


_MUL_ROWS = 256
_TR = _MUL_ROWS // 8
_TC = NV // 128
_BLK = _MUL_ROWS * NV


def _mul_body(m1, m2, kt, out, kl_o):
    i32 = jnp.int32
    w1 = m1[...]
    v = m2[...]
    w_f = lax.bitcast_convert_type(v << i32(16), jnp.float32)
    pi_f = lax.bitcast_convert_type(v & i32(-65536), jnp.float32)
    eff = (jnp.where(w1 != 0.0, 1.0, 0.0) + pi_f) * (w1 + w_f)
    out[...] = (eff.reshape(_TR, _TC, 8, 128)
                .transpose(0, 2, 1, 3)
                .reshape(_MUL_ROWS, NV))

    @pl.when(pl.program_id(0) == 0)
    def _():
        kl_o[0] = jnp.sum(kt[...])


def _mul(m1, m2, kt):
    grid = NN // _BLK
    flat_spec = pl.BlockSpec((_BLK,), lambda i: (i,))
    return pl.pallas_call(
        _mul_body,
        grid=(grid,),
        in_specs=[flat_spec, flat_spec,
                  pl.BlockSpec((ND // 128, 128), lambda i: (0, 0))],
        out_specs=(
            pl.BlockSpec((_MUL_ROWS, NV), lambda i: (i, 0)),
            pl.BlockSpec(memory_space=pltpu.MemorySpace.SMEM),
        ),
        out_shape=(
            jax.ShapeDtypeStruct((NV, NV), jnp.float32),
            jax.ShapeDtypeStruct((1,), jnp.float32),
        ),
    )(m1, m2, kt)


def kernel(theta_hard, w_disc, a_raw, b_raw, prior_a, prior_b,
           hard_idx, disc_idx):
    i32 = jnp.int32
    hr = hard_idx[:, 0].astype(i32).reshape(NH // 128, 128)
    hc = hard_idx[:, 1].astype(i32).reshape(NH // 128, 128)
    dr = disc_idx[:, 0].astype(i32).reshape(ND // 128, 128)
    dc = disc_idx[:, 1].astype(i32).reshape(ND // 128, 128)
    theta = theta_hard.reshape(NH // 128, 128)
    ar = a_raw.reshape(ND // 128, 128)
    br = b_raw.reshape(ND // 128, 128)

    a = jax.nn.softplus(a_raw) + 0.001
    b = jax.nn.softplus(b_raw) + 0.001
    kt = (gammaln(prior_a) + gammaln(prior_b) - gammaln(prior_a + prior_b)
          - gammaln(a) - gammaln(b) + gammaln(a + b)
          + (a - prior_a) * digamma(a)
          + (b - prior_b) * digamma(b)
          + (prior_a + prior_b - a - b) * digamma(a + b))

    hw, pk, hf, df = _prep(theta, ar, br, w_disc.reshape(ND // 128, 128),
                           hr, hc, dr, dc)

    m1 = jax.new_ref(jnp.zeros((NN,), jnp.float32))
    _sc_scatter1(NH, jnp.float32)(hf.reshape(NH), hw.reshape(NH), m1)
    m2 = jax.new_ref(jnp.zeros((NN,), jnp.int32))
    _sc_scatter1(ND, jnp.int32)(df.reshape(ND), pk.reshape(ND), m2)

    eff, kl = _mul(m1[...], m2[...], kt.reshape(ND // 128, 128))
    return eff, kl[0]
